# baseline (speedup 1.0000x reference)
import jax
import jax.numpy as jnp
from jax.experimental import pallas as pl
from jax.experimental.pallas import tpu as pltpu

NUM_EXPERTS = 8
BLOCK_N = 512


def _moe_block_kernel(x_ref, gw_ref, gb_ref, ew_ref, eb_ref, o_ref):
    x = x_ref[...]
    logits = jnp.dot(x, gw_ref[...], preferred_element_type=jnp.float32)
    logits = logits + gb_ref[...]
    m = jnp.max(logits, axis=-1, keepdims=True)
    p = jnp.exp(logits - m)
    g = p / jnp.sum(p, axis=-1, keepdims=True)

    acc = jnp.dot(g, eb_ref[...], preferred_element_type=jnp.float32)
    xb = x.astype(jnp.bfloat16)
    for e in range(NUM_EXPERTS):
        y = jnp.dot(xb, ew_ref[e], preferred_element_type=jnp.float32)
        acc = acc + g[:, e : e + 1] * y
    o_ref[...] = acc


@jax.jit
def kernel(x, gate_w, gate_b, expert_w, expert_b):
    n, d = x.shape
    e, _, f = expert_w.shape
    gate_b2 = gate_b.reshape(1, e)
    ew_bf16 = expert_w.astype(jnp.bfloat16)
    grid = (n // BLOCK_N,)
    return pl.pallas_call(
        _moe_block_kernel,
        grid=grid,
        in_specs=[
            pl.BlockSpec((BLOCK_N, d), lambda i: (i, 0)),
            pl.BlockSpec((d, e), lambda i: (0, 0)),
            pl.BlockSpec((1, e), lambda i: (0, 0)),
            pl.BlockSpec((e, d, f), lambda i: (0, 0, 0)),
            pl.BlockSpec((e, f), lambda i: (0, 0)),
        ],
        out_specs=pl.BlockSpec((BLOCK_N, f), lambda i: (i, 0)),
        out_shape=jax.ShapeDtypeStruct((n, f), jnp.float32),
        compiler_params=pltpu.CompilerParams(
            dimension_semantics=("parallel",),
        ),
    )(x, gate_w, gate_b2, ew_bf16, expert_b)


# speedup vs baseline: 1.1055x; 1.1055x over previous
import jax
import jax.numpy as jnp
from jax.experimental import pallas as pl
from jax.experimental.pallas import tpu as pltpu

NUM_EXPERTS = 8
BLOCK_N = 1024


def _moe_block_kernel(x_ref, gw_ref, gb_ref, ew_ref, eb_ref, o_ref):
    x = x_ref[...]
    logits = jnp.dot(x, gw_ref[...], preferred_element_type=jnp.float32)
    logits = logits + gb_ref[...]
    m = jnp.max(logits, axis=-1, keepdims=True)
    p = jnp.exp(logits - m)
    g = p / jnp.sum(p, axis=-1, keepdims=True)

    acc = jnp.dot(g, eb_ref[...], preferred_element_type=jnp.float32)
    for e in range(NUM_EXPERTS):
        y = jnp.dot(x, ew_ref[e], preferred_element_type=jnp.float32)
        acc = acc + g[:, e : e + 1] * y
    o_ref[...] = acc


@jax.jit
def kernel(x, gate_w, gate_b, expert_w, expert_b):
    n, d = x.shape
    e, _, f = expert_w.shape
    gate_b2 = gate_b.reshape(1, e)
    grid = (n // BLOCK_N,)
    return pl.pallas_call(
        _moe_block_kernel,
        grid=grid,
        in_specs=[
            pl.BlockSpec((BLOCK_N, d), lambda i: (i, 0)),
            pl.BlockSpec((d, e), lambda i: (0, 0)),
            pl.BlockSpec((1, e), lambda i: (0, 0)),
            pl.BlockSpec((e, d, f), lambda i: (0, 0, 0)),
            pl.BlockSpec((e, f), lambda i: (0, 0)),
        ],
        out_specs=pl.BlockSpec((BLOCK_N, f), lambda i: (i, 0)),
        out_shape=jax.ShapeDtypeStruct((n, f), jnp.float32),
        compiler_params=pltpu.CompilerParams(
            dimension_semantics=("parallel",),
        ),
    )(x, gate_w, gate_b2, expert_w, expert_b)
